# Initial kernel scaffold; baseline (speedup 1.0000x reference)
import jax
import jax.numpy as jnp
from jax.experimental import pallas as pl


def kernel(input_ids, emb, W, b):
    raise NotImplementedError("write your pallas kernel here")


# speedup vs baseline: 5.8630x; 5.8630x over previous
import jax
import jax.numpy as jnp
from jax import lax
from jax.experimental import pallas as pl


def _tc_body(ids_ref, emb_ref, w_ref, b_ref, out_ref):
    f32 = jnp.float32
    table = lax.dot_general(
        emb_ref[...], w_ref[...], (((1,), (1,)), ((), ())),
        preferred_element_type=f32) + b_ref[...]

    p16 = lax.broadcasted_iota(jnp.int32, (128, 16), 0) % 16
    v16 = lax.broadcasted_iota(jnp.int32, (128, 16), 1)
    left = (p16 == v16).astype(f32)
    q16 = lax.broadcasted_iota(jnp.int32, (16, 128), 1) % 16
    e16 = lax.broadcasted_iota(jnp.int32, (16, 128), 0)
    right = (q16 == e16).astype(f32)
    tiled = jnp.dot(jnp.dot(left, table, preferred_element_type=f32),
                    right, preferred_element_type=f32)
    pg = lax.broadcasted_iota(jnp.int32, (128, 128), 0) // 16
    qg = lax.broadcasted_iota(jnp.int32, (128, 128), 1) // 16
    bigT = jnp.where(pg == qg, tiled, 0.0)

    rep = (lax.broadcasted_iota(jnp.int32, (8, 128), 1) // 16
           == lax.broadcasted_iota(jnp.int32, (8, 128), 0)).astype(f32)
    y = jnp.dot(ids_ref[...].astype(f32), rep, preferred_element_type=f32)
    lmod = (lax.broadcasted_iota(jnp.int32, y.shape, 1) % 16).astype(f32)
    oh = (y == lmod).astype(f32)
    out_ref[...] = jnp.dot(oh, bigT, preferred_element_type=f32)


def kernel(input_ids, emb, W, b):
    B, T = input_ids.shape
    n = B * T
    rows = n // 8
    ids8 = input_ids.reshape(rows, 8)
    RB = 2048
    out2 = pl.pallas_call(
        _tc_body,
        grid=(rows // RB,),
        in_specs=[
            pl.BlockSpec((RB, 8), lambda i: (i, 0)),
            pl.BlockSpec((16, 16), lambda i: (0, 0)),
            pl.BlockSpec((16, 16), lambda i: (0, 0)),
            pl.BlockSpec((1, 16), lambda i: (0, 0)),
        ],
        out_specs=pl.BlockSpec((RB, 128), lambda i: (i, 0)),
        out_shape=jax.ShapeDtypeStruct((rows, 128), jnp.float32),
    )(ids8, emb, W, b.reshape(1, 16))
    return out2.reshape(B, T, 16)
